# baseline (speedup 1.0000x reference)
import jax
import jax.numpy as jnp
from jax import lax
from jax.experimental import pallas as pl
from jax.experimental.pallas import tpu as pltpu
from jax.experimental.pallas import tpu_sc as plsc

N = 10000
NP = 10240
E = 320000
ET = E + N
CHUNK = 128
SUB = 128
NC, NS = 2, 16
NW = NC * NS
NCHUNKS = 2 * (-(-ET // (2 * CHUNK * NW)))
ETP = NCHUNKS * CHUNK * NW
NSEG = ETP // SUB
ROWS_PT = NP // NS
ND = NP // 16
NDPT = ND // NS
PAD_NODE = N + 16
NOUT = 10112
DIM_OUT = 64

_f32 = jnp.float32
_i32 = jnp.int32


def _mesh():
    return plsc.VectorSubcoreMesh(
        core_axis_name="c", subcore_axis_name="s", num_cores=NC, num_subcores=NS
    )


_SC_PARAMS = pltpu.CompilerParams(needs_layout_passes=False)


def _rden_body(d0_ref, d1_ref, o_ref):
    o_ref[...] = 1.0 / (d0_ref[...] + d1_ref[...] + 1e-16)


def _tc_rden(densum):
    return pl.pallas_call(
        _rden_body,
        grid=(1,),
        in_specs=[
            pl.BlockSpec((ND, 128), lambda i: (0, 0)),
            pl.BlockSpec((ND, 128), lambda i: (0, 0)),
        ],
        out_specs=pl.BlockSpec((ND, 128), lambda i: (0, 0)),
        out_shape=jax.ShapeDtypeStruct((ND, 128), _f32),
    )(densum[0], densum[1])


def _expand_rden_glue(rden_packed, H):
    r = rden_packed.reshape(NP, 8)[:, :H]
    return jnp.broadcast_to(r[:, :, None], (NP, H, 128 // H)).reshape(NP, 128)


def _mm_first_body(x_ref, wl_ref, bl_ref, wr_ref, br_ref, xl_ref, xr_ref):
    h = x_ref[...]
    xl_ref[...] = jnp.dot(h, wl_ref[...], preferred_element_type=_f32) + bl_ref[...]
    xr_ref[...] = jnp.dot(h, wr_ref[...], preferred_element_type=_f32) + br_ref[...]


def _mm_mid_body(o0_ref, o1_ref, rden_ref, bo_ref, wl_ref, bl_ref,
                 wr_ref, br_ref, xl_ref, xr_ref):
    hin = (o0_ref[...] + o1_ref[...]) * rden_ref[...] + bo_ref[...]
    h = jnp.where(hin > 0.0, hin, jnp.exp(jnp.minimum(hin, 0.0)) - 1.0)
    xl_ref[...] = jnp.dot(h, wl_ref[...], preferred_element_type=_f32) + bl_ref[...]
    xr_ref[...] = jnp.dot(h, wr_ref[...], preferred_element_type=_f32) + br_ref[...]


def _tc_matmul_first(x, wl, bl, wr, br):
    hc = wl.shape[1]
    blk = 1024
    return pl.pallas_call(
        _mm_first_body,
        grid=(NP // blk,),
        in_specs=[
            pl.BlockSpec((blk, x.shape[1]), lambda i: (i, 0)),
            pl.BlockSpec((x.shape[1], hc), lambda i: (0, 0)),
            pl.BlockSpec((hc,), lambda i: (0,)),
            pl.BlockSpec((x.shape[1], hc), lambda i: (0, 0)),
            pl.BlockSpec((hc,), lambda i: (0,)),
        ],
        out_specs=[
            pl.BlockSpec((blk, hc), lambda i: (i, 0)),
            pl.BlockSpec((blk, hc), lambda i: (i, 0)),
        ],
        out_shape=[
            jax.ShapeDtypeStruct((NP, hc), _f32),
            jax.ShapeDtypeStruct((NP, hc), _f32),
        ],
    )(x, wl, bl, wr, br)


def _tc_matmul_mid(o0, o1, rden_full, bo, wl, bl, wr, br):
    hc = wl.shape[1]
    blk = 1024
    return pl.pallas_call(
        _mm_mid_body,
        grid=(NP // blk,),
        in_specs=[
            pl.BlockSpec((blk, 128), lambda i: (i, 0)),
            pl.BlockSpec((blk, 128), lambda i: (i, 0)),
            pl.BlockSpec((blk, 128), lambda i: (i, 0)),
            pl.BlockSpec((128,), lambda i: (0,)),
            pl.BlockSpec((128, hc), lambda i: (0, 0)),
            pl.BlockSpec((hc,), lambda i: (0,)),
            pl.BlockSpec((128, hc), lambda i: (0, 0)),
            pl.BlockSpec((hc,), lambda i: (0,)),
        ],
        out_specs=[
            pl.BlockSpec((blk, hc), lambda i: (i, 0)),
            pl.BlockSpec((blk, hc), lambda i: (i, 0)),
        ],
        out_shape=[
            jax.ShapeDtypeStruct((NP, hc), _f32),
            jax.ShapeDtypeStruct((NP, hc), _f32),
        ],
    )(o0, o1, rden_full, bo, wl, bl, wr, br)


def _fin_body(o0_ref, o1_ref, rden_ref, bo_ref, out_ref):
    x = ((o0_ref[...] + o1_ref[...]) * rden_ref[...])[:, :DIM_OUT] + bo_ref[...]
    m = jnp.max(x, axis=1, keepdims=True)
    s = jnp.log(jnp.sum(jnp.exp(x - m), axis=1, keepdims=True))
    out_ref[...] = x - m - s


def _tc_final(o0, o1, rden_full, bo):
    blk = 1024
    return pl.pallas_call(
        _fin_body,
        grid=(NP // blk,),
        in_specs=[
            pl.BlockSpec((blk, 128), lambda i: (i, 0)),
            pl.BlockSpec((blk, 128), lambda i: (i, 0)),
            pl.BlockSpec((blk, 128), lambda i: (i, 0)),
            pl.BlockSpec((DIM_OUT,), lambda i: (0,)),
        ],
        out_specs=pl.BlockSpec((blk, DIM_OUT), lambda i: (i, 0)),
        out_shape=jax.ShapeDtypeStruct((NP, DIM_OUT), _f32),
    )(o0, o1, rden_full, bo)


def _sc_pass1(H, C):
    HC = H * C
    assert HC == 128 and CHUNK == SUB

    def body(xl_hbm, xr_hbm, srcdst_hbm, att_hbm, zerosf_hbm,
             ex_hbm, densum_hbm,
             xl0, xl1, xr0, xr1, exb0, exb1, exc_t, sdx, didxq,
             att_v, sp_den, sem0, sem1, sem_s, sem_e):
        xlr = (xl0, xl1)
        xrr = (xr0, xr1)
        exb = (exb0, exb1)
        sems = (sem0, sem1)
        cid = lax.axis_index("c")
        sid = lax.axis_index("s")
        wid = cid * NS + sid
        d0 = sid * NDPT
        pltpu.sync_copy(zerosf_hbm.at[pl.ds(0, NDPT)], exb0.at[pl.ds(0, NDPT)])
        pltpu.sync_copy(exb0.at[pl.ds(0, NDPT)], sp_den.at[pl.ds(d0, NDPT)])
        pltpu.sync_copy(zerosf_hbm.at[pl.ds(0, SUB)], exb0)
        pltpu.sync_copy(zerosf_hbm.at[pl.ds(0, SUB)], exb1)
        pltpu.sync_copy(att_hbm, att_v)
        plsc.subcore_barrier()

        def batchload(ci):
            seg = wid * NCHUNKS + ci
            pltpu.sync_copy(srcdst_hbm.at[pl.ds(seg, 16)],
                            sdx.at[(ci // 16) % 2])

        def fire(ci, b):
            sl = sdx.at[(ci // 16) % 2, ci % 16]
            pltpu.async_copy(xl_hbm.at[sl.at[0]], xlr[b], sems[b])
            pltpu.async_copy(xr_hbm.at[sl.at[1]], xrr[b], sems[b])

        def drain(b):
            pltpu.make_async_copy(xl_hbm.at[pl.ds(0, SUB)], xlr[b], sems[b]).wait()
            pltpu.make_async_copy(xr_hbm.at[pl.ds(0, SUB)], xrr[b], sems[b]).wait()

        def dstv_of(ci, gi):
            return sdx[(ci // 16) % 2, ci % 16, 1, pl.ds(gi * 16, 16)]

        batchload(0)
        fire(0, 0)

        def pair_body(t, _):
            ci0 = t * 2
            for b in range(2):
                ci = ci0 + b
                nb = 1 - b
                nci = jnp.minimum(ci + 1, NCHUNKS - 1)

                @pl.when(nci % 16 == 0)
                def _load():
                    batchload(nci)

                fire(nci, nb)
                drain(b)

                @pl.when(ci >= 2)
                def _retire():
                    pltpu.make_async_copy(
                        zerosf_hbm.at[pl.ds(0, SUB)], exb[b], sem_s).wait()
                    pltpu.make_async_copy(
                        ex_hbm.at[0], exc_t.at[b], sem_e).wait()

                    def rezero(gi, _g):
                        dstv = dstv_of(ci - 2, gi)
                        rowv = gi * 16 + lax.iota(_i32, 16)
                        colbase = (dstv & 15) * 8
                        zv = jnp.zeros((16,), _f32)
                        for h in range(H):
                            plsc.store_scatter(exb[b], [rowv, colbase + h], zv)
                        return 0

                    lax.fori_loop(0, CHUNK // 16, rezero, 0)

                seg = wid * NCHUNKS + ci

                def group(gi, _g):
                    lane = lax.iota(_i32, 16)
                    rowv = gi * 16 + lane
                    dstv = dstv_of(ci, gi)
                    colbase = (dstv & 15) * 8
                    plsc.store_scatter(
                        didxq,
                        [jnp.full((16,), b, _i32), gi * 16 + lane],
                        lax.shift_right_logical(dstv, 4))
                    for h in range(H):
                        def cstep(c2, acc):
                            blk = c2 // 16
                            cl = c2 % 16
                            colv = (h * C + blk * 16) + ((cl + lane) & 15)
                            xlv = plsc.load_gather(xlr[b], [rowv, colv])
                            xrv = plsc.load_gather(xrr[b], [rowv, colv])
                            v = xlv + xrv
                            lv = jnp.where(v > 0.0, v, v * 0.2)
                            bg = h * C // 16 + blk
                            av = att_v[pl.ds(bg * 32 + cl, 16)]
                            return acc + av * lv
                        acc = lax.fori_loop(0, C, cstep,
                                            jnp.zeros((16,), _f32), unroll=8)
                        exh = jnp.exp(acc)
                        plsc.store_scatter(
                            exc_t,
                            [jnp.full((16,), b, _i32),
                             jnp.full((16,), h, _i32), rowv],
                            exh)
                        plsc.store_scatter(exb[b], [rowv, colbase + h], exh)
                    return 0

                lax.fori_loop(0, CHUNK // 16, group, 0)
                pltpu.async_copy(exc_t.at[b], ex_hbm.at[seg], sem_e)
                pltpu.async_copy(exb[b], sp_den.at[didxq.at[b]], sem_s,
                                 add=True)
            return 0

        lax.fori_loop(0, NCHUNKS // 2, pair_body, 0)
        drain(0)
        for b in range(2):
            pltpu.make_async_copy(
                zerosf_hbm.at[pl.ds(0, SUB)], exb[b], sem_s).wait()
            pltpu.make_async_copy(ex_hbm.at[0], exc_t.at[b], sem_e).wait()
        plsc.subcore_barrier()
        pltpu.sync_copy(sp_den.at[pl.ds(d0, NDPT)], exb0.at[pl.ds(0, NDPT)])
        pltpu.sync_copy(exb0.at[pl.ds(0, NDPT)],
                        densum_hbm.at[cid, pl.ds(d0, NDPT)])

    return pl.kernel(
        body,
        out_type=[
            jax.ShapeDtypeStruct((NSEG, 8, CHUNK), _f32),
            jax.ShapeDtypeStruct((NC, ND, 128), _f32),
        ],
        mesh=_mesh(),
        compiler_params=_SC_PARAMS,
        scratch_types=[
            pltpu.VMEM((CHUNK, HC), _f32),
            pltpu.VMEM((CHUNK, HC), _f32),
            pltpu.VMEM((CHUNK, HC), _f32),
            pltpu.VMEM((CHUNK, HC), _f32),
            pltpu.VMEM((CHUNK, 128), _f32),
            pltpu.VMEM((CHUNK, 128), _f32),
            pltpu.VMEM((2, 8, CHUNK), _f32),
            pltpu.VMEM((2, 16, 2, SUB), _i32),
            pltpu.VMEM((2, SUB), _i32),
            pltpu.VMEM((HC * 2,), _f32),
            pltpu.VMEM_SHARED((ND, 128), _f32),
            pltpu.SemaphoreType.DMA,
            pltpu.SemaphoreType.DMA,
            pltpu.SemaphoreType.DMA,
            pltpu.SemaphoreType.DMA,
        ],
    )


def _sc_pass2(H, C):
    HC = H * C
    assert HC == 128 and CHUNK == SUB

    def body(xl_hbm, srcdst_hbm, ex_hbm, zerosf_hbm,
             outpart_hbm,
             xl0, xl1, exc_buf, sd, sp_out, sem0, sem1, sem_s):
        xlr = (xl0, xl1)
        sems = (sem0, sem1)
        cid = lax.axis_index("c")
        sid = lax.axis_index("s")
        wid = cid * NS + sid
        nslices = NOUT // SUB
        for k in range(-(-nslices // NS)):
            sl = sid + NS * k

            @pl.when(sl < nslices)
            def _zero():
                pltpu.sync_copy(zerosf_hbm.at[pl.ds(0, SUB)], xl0)
                pltpu.sync_copy(xl0, sp_out.at[pl.ds(sl * SUB, SUB)])
        plsc.subcore_barrier()

        def idxload(ci, b):
            seg = wid * NCHUNKS + ci
            pltpu.sync_copy(srcdst_hbm.at[seg], sd.at[b])

        def fire(b):
            pltpu.async_copy(xl_hbm.at[sd.at[b, 0]], xlr[b], sems[b])

        def drain(b):
            pltpu.make_async_copy(xl_hbm.at[pl.ds(0, SUB)], xlr[b], sems[b]).wait()

        idxload(0, 0)
        fire(0)

        def pair_body(t, _):
            ci0 = t * 2
            for b in range(2):
                ci = ci0 + b
                nb = 1 - b
                nci = jnp.minimum(ci + 1, NCHUNKS - 1)
                idxload(nci, nb)
                seg = wid * NCHUNKS + ci
                pltpu.sync_copy(ex_hbm.at[seg], exc_buf)
                drain(b)

                def group(gi, _g):
                    lane = lax.iota(_i32, 16)
                    rowv = gi * 16 + lane
                    for h in range(H):
                        hv = jnp.full((16,), h, _i32)
                        exv = plsc.load_gather(exc_buf, [hv, rowv])

                        def cstep(c2, _c):
                            blk = c2 // 16
                            cl = c2 % 16
                            colv = (h * C + blk * 16) + ((cl + lane) & 15)
                            xlv = plsc.load_gather(xlr[b], [rowv, colv])
                            plsc.store_scatter(xlr[b], [rowv, colv], xlv * exv)
                            return 0

                        lax.fori_loop(0, C, cstep, 0, unroll=8)
                    return 0

                lax.fori_loop(0, CHUNK // 16, group, 0)

                @pl.when(ci >= 1)
                def _retire():
                    pltpu.make_async_copy(
                        zerosf_hbm.at[pl.ds(0, SUB)], xlr[nb], sem_s).wait()

                fire(nb)
                pltpu.async_copy(xlr[b], sp_out.at[sd.at[b, 1]], sem_s,
                                 add=True)
            return 0

        lax.fori_loop(0, NCHUNKS // 2, pair_body, 0)
        drain(0)
        pltpu.make_async_copy(
            zerosf_hbm.at[pl.ds(0, SUB)], xl1, sem_s).wait()
        plsc.subcore_barrier()
        for k in range(-(-nslices // NS)):
            sl = sid + NS * k

            @pl.when(sl < nslices)
            def _readout():
                pltpu.sync_copy(sp_out.at[pl.ds(sl * SUB, SUB)], xl0)
                pltpu.sync_copy(xl0, outpart_hbm.at[cid, pl.ds(sl * SUB, SUB)])

    return pl.kernel(
        body,
        out_type=jax.ShapeDtypeStruct((NC, NP, HC), _f32),
        mesh=_mesh(),
        compiler_params=_SC_PARAMS,
        scratch_types=[
            pltpu.VMEM((CHUNK, HC), _f32),
            pltpu.VMEM((CHUNK, HC), _f32),
            pltpu.VMEM((8, CHUNK), _f32),
            pltpu.VMEM((2, 2, SUB), _i32),
            pltpu.VMEM_SHARED((NOUT, HC), _f32),
            pltpu.SemaphoreType.DMA,
            pltpu.SemaphoreType.DMA,
            pltpu.SemaphoreType.DMA,
        ],
    )


def kernel(x, edge_index, Wl1, bl1, Wr1, br1, att1, bo1,
           Wl2, bl2, Wr2, br2, att2, bo2,
           Wl3, bl3, Wr3, br3, att3, bo3,
           Wl4, bl4, Wr4, br4, att4, bo4):
    loop = jnp.arange(N, dtype=edge_index.dtype)
    pad = jnp.full((ETP - ET,), PAD_NODE, dtype=edge_index.dtype)
    padx = jnp.full(((ETP - ET) + 16 * SUB,), PAD_NODE, dtype=edge_index.dtype)
    src = jnp.concatenate([edge_index[0], loop, padx])
    dst = jnp.concatenate([edge_index[1], loop, padx])
    srcdst = jnp.stack([src.reshape(NSEG + 16, SUB),
                        dst.reshape(NSEG + 16, SUB)], axis=1)

    xp = jnp.zeros((NP, x.shape[1]), _f32).at[:N].set(x)
    zerosf = jnp.zeros((NP, 128), _f32)

    Wl4p = jnp.zeros((128, 128), _f32).at[:, :DIM_OUT].set(Wl4)
    Wr4p = jnp.zeros((128, 128), _f32).at[:, :DIM_OUT].set(Wr4)
    bl4p = jnp.zeros((128,), _f32).at[:DIM_OUT].set(bl4)
    br4p = jnp.zeros((128,), _f32).at[:DIM_OUT].set(br4)
    att4p = jnp.zeros((1, 128), _f32).at[:, :DIM_OUT].set(att4)

    layer_cfgs = [
        (8, 16, Wl1, bl1, Wr1, br1, att1, bo1),
        (8, 16, Wl2, bl2, Wr2, br2, att2, bo2),
        (8, 16, Wl3, bl3, Wr3, br3, att3, bo3),
        (1, 128, Wl4p, bl4p, Wr4p, br4p, att4p, bo4),
    ]

    o0 = o1 = None
    rden_full = None
    bo_prev = None
    for li, (H, C, Wl, bl, Wr, br, att, bo) in enumerate(layer_cfgs):
        HC = H * C
        if li == 0:
            xl, xr = _tc_matmul_first(xp, Wl, bl, Wr, br)
        else:
            xl, xr = _tc_matmul_mid(o0, o1, rden_full, bo_prev, Wl, bl, Wr, br)
        a16 = att.reshape(HC // 16, 16)
        attflat = jnp.concatenate([a16, a16], axis=1).reshape(HC * 2)
        ex, densum = _sc_pass1(H, C)(xl, xr, srcdst, attflat, zerosf)
        outpart = _sc_pass2(H, C)(xl, srcdst, ex, zerosf)
        o0, o1 = outpart[0], outpart[1]
        rden_full = _expand_rden_glue(_tc_rden(densum), H)
        bo_prev = bo

    out = _tc_final(o0, o1, rden_full, bo_prev)
    return out[:N]


# speedup vs baseline: 1.2176x; 1.2176x over previous
import jax
import jax.numpy as jnp
from jax import lax
from jax.experimental import pallas as pl
from jax.experimental.pallas import tpu as pltpu
from jax.experimental.pallas import tpu_sc as plsc

N = 10000
NP = 10240
E = 320000
ET = E + N
CHUNK = 128
SUB = 128
NC, NS = 2, 16
NW = NC * NS
NCHUNKS = 2 * (-(-ET // (2 * CHUNK * NW)))
ETP = NCHUNKS * CHUNK * NW
NSEG = ETP // SUB
ROWS_PT = NP // NS
ND = NP // 16
NDPT = ND // NS
PAD_NODE = N + 16
NOUT = 10112
DIM_OUT = 64

_f32 = jnp.float32
_i32 = jnp.int32


def _mesh():
    return plsc.VectorSubcoreMesh(
        core_axis_name="c", subcore_axis_name="s", num_cores=NC, num_subcores=NS
    )


_SC_PARAMS = pltpu.CompilerParams(needs_layout_passes=False)


def _rden_body(d0_ref, d1_ref, o_ref):
    o_ref[...] = 1.0 / (d0_ref[...] + d1_ref[...] + 1e-16)


def _tc_rden(densum):
    return pl.pallas_call(
        _rden_body,
        grid=(1,),
        in_specs=[
            pl.BlockSpec((ND, 128), lambda i: (0, 0)),
            pl.BlockSpec((ND, 128), lambda i: (0, 0)),
        ],
        out_specs=pl.BlockSpec((ND, 128), lambda i: (0, 0)),
        out_shape=jax.ShapeDtypeStruct((ND, 128), _f32),
    )(densum[0], densum[1])


def _expand_rden_glue(rden_packed, H):
    r = rden_packed.reshape(NP, 8)[:, :H]
    return jnp.broadcast_to(r[:, :, None], (NP, H, 128 // H)).reshape(NP, 128)


def _mm_first_body(x_ref, wl_ref, bl_ref, wr_ref, br_ref, xl_ref, xr_ref):
    h = x_ref[...]
    xl_ref[...] = jnp.dot(h, wl_ref[...], preferred_element_type=_f32) + bl_ref[...]
    xr_ref[...] = jnp.dot(h, wr_ref[...], preferred_element_type=_f32) + br_ref[...]


def _mm_mid_body(o0_ref, o1_ref, rden_ref, bo_ref, wl_ref, bl_ref,
                 wr_ref, br_ref, xl_ref, xr_ref):
    hin = (o0_ref[...] + o1_ref[...]) * rden_ref[...] + bo_ref[...]
    h = jnp.where(hin > 0.0, hin, jnp.exp(jnp.minimum(hin, 0.0)) - 1.0)
    xl_ref[...] = jnp.dot(h, wl_ref[...], preferred_element_type=_f32) + bl_ref[...]
    xr_ref[...] = jnp.dot(h, wr_ref[...], preferred_element_type=_f32) + br_ref[...]


def _tc_matmul_first(x, wl, bl, wr, br):
    hc = wl.shape[1]
    blk = 1024
    return pl.pallas_call(
        _mm_first_body,
        grid=(NP // blk,),
        in_specs=[
            pl.BlockSpec((blk, x.shape[1]), lambda i: (i, 0)),
            pl.BlockSpec((x.shape[1], hc), lambda i: (0, 0)),
            pl.BlockSpec((hc,), lambda i: (0,)),
            pl.BlockSpec((x.shape[1], hc), lambda i: (0, 0)),
            pl.BlockSpec((hc,), lambda i: (0,)),
        ],
        out_specs=[
            pl.BlockSpec((blk, hc), lambda i: (i, 0)),
            pl.BlockSpec((blk, hc), lambda i: (i, 0)),
        ],
        out_shape=[
            jax.ShapeDtypeStruct((NP, hc), _f32),
            jax.ShapeDtypeStruct((NP, hc), _f32),
        ],
    )(x, wl, bl, wr, br)


def _tc_matmul_mid(o0, o1, rden_full, bo, wl, bl, wr, br):
    hc = wl.shape[1]
    blk = 1024
    return pl.pallas_call(
        _mm_mid_body,
        grid=(NP // blk,),
        in_specs=[
            pl.BlockSpec((blk, 128), lambda i: (i, 0)),
            pl.BlockSpec((blk, 128), lambda i: (i, 0)),
            pl.BlockSpec((blk, 128), lambda i: (i, 0)),
            pl.BlockSpec((128,), lambda i: (0,)),
            pl.BlockSpec((128, hc), lambda i: (0, 0)),
            pl.BlockSpec((hc,), lambda i: (0,)),
            pl.BlockSpec((128, hc), lambda i: (0, 0)),
            pl.BlockSpec((hc,), lambda i: (0,)),
        ],
        out_specs=[
            pl.BlockSpec((blk, hc), lambda i: (i, 0)),
            pl.BlockSpec((blk, hc), lambda i: (i, 0)),
        ],
        out_shape=[
            jax.ShapeDtypeStruct((NP, hc), _f32),
            jax.ShapeDtypeStruct((NP, hc), _f32),
        ],
    )(o0, o1, rden_full, bo, wl, bl, wr, br)


def _fin_body(o0_ref, o1_ref, rden_ref, bo_ref, out_ref):
    x = ((o0_ref[...] + o1_ref[...]) * rden_ref[...])[:, :DIM_OUT] + bo_ref[...]
    m = jnp.max(x, axis=1, keepdims=True)
    s = jnp.log(jnp.sum(jnp.exp(x - m), axis=1, keepdims=True))
    out_ref[...] = x - m - s


def _tc_final(o0, o1, rden_full, bo):
    blk = 1024
    return pl.pallas_call(
        _fin_body,
        grid=(NP // blk,),
        in_specs=[
            pl.BlockSpec((blk, 128), lambda i: (i, 0)),
            pl.BlockSpec((blk, 128), lambda i: (i, 0)),
            pl.BlockSpec((blk, 128), lambda i: (i, 0)),
            pl.BlockSpec((DIM_OUT,), lambda i: (0,)),
        ],
        out_specs=pl.BlockSpec((blk, DIM_OUT), lambda i: (i, 0)),
        out_shape=jax.ShapeDtypeStruct((NP, DIM_OUT), _f32),
    )(o0, o1, rden_full, bo)


def _sc_pass1(H, C):
    HC = H * C
    assert HC == 128 and CHUNK == SUB

    def body(xl_hbm, xr_hbm, srcdst_hbm, att_hbm, zerosf_hbm,
             ex_hbm, densum_hbm,
             xl0, xl1, xr0, xr1, exb0, exb1, exc_t, sdx, didxq,
             att_v, sp_den, sem0, sem1, sem_s, sem_e):
        xlr = (xl0, xl1)
        xrr = (xr0, xr1)
        exb = (exb0, exb1)
        sems = (sem0, sem1)
        cid = lax.axis_index("c")
        sid = lax.axis_index("s")
        wid = cid * NS + sid
        d0 = sid * NDPT
        pltpu.sync_copy(zerosf_hbm.at[pl.ds(0, NDPT)], exb0.at[pl.ds(0, NDPT)])
        pltpu.sync_copy(exb0.at[pl.ds(0, NDPT)], sp_den.at[pl.ds(d0, NDPT)])
        pltpu.sync_copy(zerosf_hbm.at[pl.ds(0, SUB)], exb0)
        pltpu.sync_copy(zerosf_hbm.at[pl.ds(0, SUB)], exb1)
        pltpu.sync_copy(att_hbm, att_v)
        plsc.subcore_barrier()

        def batchload(ci):
            seg = wid * NCHUNKS + ci
            pltpu.sync_copy(srcdst_hbm.at[pl.ds(seg, 16)],
                            sdx.at[(ci // 16) % 2])

        def fire(ci, b):
            sl = sdx.at[(ci // 16) % 2, ci % 16]
            pltpu.async_copy(xl_hbm.at[sl.at[0]], xlr[b], sems[b])
            pltpu.async_copy(xr_hbm.at[sl.at[1]], xrr[b], sems[b])

        def drain(b):
            pltpu.make_async_copy(xl_hbm.at[pl.ds(0, SUB)], xlr[b], sems[b]).wait()
            pltpu.make_async_copy(xr_hbm.at[pl.ds(0, SUB)], xrr[b], sems[b]).wait()

        def dstv_of(ci, gi):
            return sdx[(ci // 16) % 2, ci % 16, 1, pl.ds(gi * 16, 16)]

        batchload(0)
        fire(0, 0)

        def pair_body(t, _):
            ci0 = t * 2
            for b in range(2):
                ci = ci0 + b
                nb = 1 - b
                nci = jnp.minimum(ci + 1, NCHUNKS - 1)

                @pl.when(nci % 16 == 0)
                def _load():
                    batchload(nci)

                fire(nci, nb)
                drain(b)

                @pl.when(ci >= 2)
                def _retire():
                    pltpu.make_async_copy(
                        zerosf_hbm.at[pl.ds(0, SUB)], exb[b], sem_s).wait()
                    pltpu.make_async_copy(
                        ex_hbm.at[0], exc_t.at[b], sem_e).wait()

                    def rezero(gi, _g):
                        dstv = dstv_of(ci - 2, gi)
                        rowv = gi * 16 + lax.iota(_i32, 16)
                        colbase = (dstv & 15) * 8
                        zv = jnp.zeros((16,), _f32)
                        for h in range(H):
                            plsc.store_scatter(exb[b], [rowv, colbase + h], zv)
                        return 0

                    lax.fori_loop(0, CHUNK // 16, rezero, 0)

                seg = wid * NCHUNKS + ci

                def group(gi, _g):
                    lane = lax.iota(_i32, 16)
                    rowv = gi * 16 + lane
                    dstv = dstv_of(ci, gi)
                    colbase = (dstv & 15) * 8
                    plsc.store_scatter(
                        didxq,
                        [jnp.full((16,), b, _i32), gi * 16 + lane],
                        lax.shift_right_logical(dstv, 4))
                    for h in range(H):
                        def cstep(c2, acc):
                            blk = c2 // 16
                            cl = c2 % 16
                            colv = (h * C + blk * 16) + ((cl + lane) & 15)
                            xlv = plsc.load_gather(xlr[b], [rowv, colv])
                            xrv = plsc.load_gather(xrr[b], [rowv, colv])
                            v = xlv + xrv
                            lv = jnp.where(v > 0.0, v, v * 0.2)
                            bg = h * C // 16 + blk
                            av = att_v[pl.ds(bg * 32 + cl, 16)]
                            return acc + av * lv
                        acc = lax.fori_loop(0, C, cstep,
                                            jnp.zeros((16,), _f32), unroll=8)
                        exh = jnp.exp(acc)
                        plsc.store_scatter(
                            exc_t,
                            [jnp.full((16,), b, _i32),
                             jnp.full((16,), h, _i32), rowv],
                            exh)
                        plsc.store_scatter(exb[b], [rowv, colbase + h], exh)
                    return 0

                lax.fori_loop(0, CHUNK // 16, group, 0)
                pltpu.async_copy(exc_t.at[b], ex_hbm.at[seg], sem_e)
                pltpu.async_copy(exb[b], sp_den.at[didxq.at[b]], sem_s,
                                 add=True)
            return 0

        lax.fori_loop(0, NCHUNKS // 2, pair_body, 0)
        drain(0)
        for b in range(2):
            pltpu.make_async_copy(
                zerosf_hbm.at[pl.ds(0, SUB)], exb[b], sem_s).wait()
            pltpu.make_async_copy(ex_hbm.at[0], exc_t.at[b], sem_e).wait()
        plsc.subcore_barrier()
        pltpu.sync_copy(sp_den.at[pl.ds(d0, NDPT)], exb0.at[pl.ds(0, NDPT)])
        pltpu.sync_copy(exb0.at[pl.ds(0, NDPT)],
                        densum_hbm.at[cid, pl.ds(d0, NDPT)])

    return pl.kernel(
        body,
        out_type=[
            jax.ShapeDtypeStruct((NSEG, 8, CHUNK), _f32),
            jax.ShapeDtypeStruct((NC, ND, 128), _f32),
        ],
        mesh=_mesh(),
        compiler_params=_SC_PARAMS,
        scratch_types=[
            pltpu.VMEM((CHUNK, HC), _f32),
            pltpu.VMEM((CHUNK, HC), _f32),
            pltpu.VMEM((CHUNK, HC), _f32),
            pltpu.VMEM((CHUNK, HC), _f32),
            pltpu.VMEM((CHUNK, 128), _f32),
            pltpu.VMEM((CHUNK, 128), _f32),
            pltpu.VMEM((2, 8, CHUNK), _f32),
            pltpu.VMEM((2, 16, 2, SUB), _i32),
            pltpu.VMEM((2, SUB), _i32),
            pltpu.VMEM((HC * 2,), _f32),
            pltpu.VMEM_SHARED((ND, 128), _f32),
            pltpu.SemaphoreType.DMA,
            pltpu.SemaphoreType.DMA,
            pltpu.SemaphoreType.DMA,
            pltpu.SemaphoreType.DMA,
        ],
    )


def _sc_pass2(H, C):
    HC = H * C
    assert HC == 128 and CHUNK == SUB

    def body(xl_hbm, srcdst_hbm, ex_hbm, zerosf_hbm,
             outpart_hbm,
             xl0, xl1, exc_buf, sd, sp_out, sem0, sem1, sem_s):
        xlr = (xl0, xl1)
        sems = (sem0, sem1)
        cid = lax.axis_index("c")
        sid = lax.axis_index("s")
        wid = cid * NS + sid
        nslices = NOUT // SUB
        for k in range(-(-nslices // NS)):
            sl = sid + NS * k

            @pl.when(sl < nslices)
            def _zero():
                pltpu.sync_copy(zerosf_hbm.at[pl.ds(0, SUB)], xl0)
                pltpu.sync_copy(xl0, sp_out.at[pl.ds(sl * SUB, SUB)])
        plsc.subcore_barrier()

        def idxload(ci, b):
            seg = wid * NCHUNKS + ci
            pltpu.sync_copy(srcdst_hbm.at[seg], sd.at[b])

        def fire(b):
            pltpu.async_copy(xl_hbm.at[sd.at[b, 0]], xlr[b], sems[b])

        def drain(b):
            pltpu.make_async_copy(xl_hbm.at[pl.ds(0, SUB)], xlr[b], sems[b]).wait()

        idxload(0, 0)
        fire(0)

        def pair_body(t, _):
            ci0 = t * 2
            for b in range(2):
                ci = ci0 + b
                nb = 1 - b
                nci = jnp.minimum(ci + 1, NCHUNKS - 1)
                idxload(nci, nb)

                @pl.when(ci >= 1)
                def _retire():
                    pltpu.make_async_copy(
                        zerosf_hbm.at[pl.ds(0, SUB)], xlr[nb], sem_s).wait()

                fire(nb)
                seg = wid * NCHUNKS + ci
                pltpu.sync_copy(ex_hbm.at[seg], exc_buf)
                drain(b)

                def group(gi, _g):
                    lane = lax.iota(_i32, 16)
                    rowv = gi * 16 + lane
                    for h in range(H):
                        hv = jnp.full((16,), h, _i32)
                        exv = plsc.load_gather(exc_buf, [hv, rowv])

                        def cstep(c2, _c):
                            blk = c2 // 16
                            cl = c2 % 16
                            colv = (h * C + blk * 16) + ((cl + lane) & 15)
                            xlv = plsc.load_gather(xlr[b], [rowv, colv])
                            plsc.store_scatter(xlr[b], [rowv, colv], xlv * exv)
                            return 0

                        lax.fori_loop(0, C, cstep, 0, unroll=8)
                    return 0

                lax.fori_loop(0, CHUNK // 16, group, 0)
                pltpu.async_copy(xlr[b], sp_out.at[sd.at[b, 1]], sem_s,
                                 add=True)
            return 0

        lax.fori_loop(0, NCHUNKS // 2, pair_body, 0)
        drain(0)
        pltpu.make_async_copy(
            zerosf_hbm.at[pl.ds(0, SUB)], xl1, sem_s).wait()
        plsc.subcore_barrier()
        for k in range(-(-nslices // NS)):
            sl = sid + NS * k

            @pl.when(sl < nslices)
            def _readout():
                pltpu.sync_copy(sp_out.at[pl.ds(sl * SUB, SUB)], xl0)
                pltpu.sync_copy(xl0, outpart_hbm.at[cid, pl.ds(sl * SUB, SUB)])

    return pl.kernel(
        body,
        out_type=jax.ShapeDtypeStruct((NC, NP, HC), _f32),
        mesh=_mesh(),
        compiler_params=_SC_PARAMS,
        scratch_types=[
            pltpu.VMEM((CHUNK, HC), _f32),
            pltpu.VMEM((CHUNK, HC), _f32),
            pltpu.VMEM((8, CHUNK), _f32),
            pltpu.VMEM((2, 2, SUB), _i32),
            pltpu.VMEM_SHARED((NOUT, HC), _f32),
            pltpu.SemaphoreType.DMA,
            pltpu.SemaphoreType.DMA,
            pltpu.SemaphoreType.DMA,
        ],
    )


def kernel(x, edge_index, Wl1, bl1, Wr1, br1, att1, bo1,
           Wl2, bl2, Wr2, br2, att2, bo2,
           Wl3, bl3, Wr3, br3, att3, bo3,
           Wl4, bl4, Wr4, br4, att4, bo4):
    loop = jnp.arange(N, dtype=edge_index.dtype)
    pad = jnp.full((ETP - ET,), PAD_NODE, dtype=edge_index.dtype)
    padx = jnp.full(((ETP - ET) + 16 * SUB,), PAD_NODE, dtype=edge_index.dtype)
    src = jnp.concatenate([edge_index[0], loop, padx])
    dst = jnp.concatenate([edge_index[1], loop, padx])
    srcdst = jnp.stack([src.reshape(NSEG + 16, SUB),
                        dst.reshape(NSEG + 16, SUB)], axis=1)

    xp = jnp.zeros((NP, x.shape[1]), _f32).at[:N].set(x)
    zerosf = jnp.zeros((NP, 128), _f32)

    Wl4p = jnp.zeros((128, 128), _f32).at[:, :DIM_OUT].set(Wl4)
    Wr4p = jnp.zeros((128, 128), _f32).at[:, :DIM_OUT].set(Wr4)
    bl4p = jnp.zeros((128,), _f32).at[:DIM_OUT].set(bl4)
    br4p = jnp.zeros((128,), _f32).at[:DIM_OUT].set(br4)
    att4p = jnp.zeros((1, 128), _f32).at[:, :DIM_OUT].set(att4)

    layer_cfgs = [
        (8, 16, Wl1, bl1, Wr1, br1, att1, bo1),
        (8, 16, Wl2, bl2, Wr2, br2, att2, bo2),
        (8, 16, Wl3, bl3, Wr3, br3, att3, bo3),
        (1, 128, Wl4p, bl4p, Wr4p, br4p, att4p, bo4),
    ]

    o0 = o1 = None
    rden_full = None
    bo_prev = None
    for li, (H, C, Wl, bl, Wr, br, att, bo) in enumerate(layer_cfgs):
        HC = H * C
        if li == 0:
            xl, xr = _tc_matmul_first(xp, Wl, bl, Wr, br)
        else:
            xl, xr = _tc_matmul_mid(o0, o1, rden_full, bo_prev, Wl, bl, Wr, br)
        a16 = att.reshape(HC // 16, 16)
        attflat = jnp.concatenate([a16, a16], axis=1).reshape(HC * 2)
        ex, densum = _sc_pass1(H, C)(xl, xr, srcdst, attflat, zerosf)
        outpart = _sc_pass2(H, C)(xl, srcdst, ex, zerosf)
        o0, o1 = outpart[0], outpart[1]
        rden_full = _expand_rden_glue(_tc_rden(densum), H)
        bo_prev = bo

    out = _tc_final(o0, o1, rden_full, bo_prev)
    return out[:N]
